# initial kernel scaffold (unmeasured)
import jax
import jax.numpy as jnp
from jax import lax
from jax.experimental import pallas as pl
from jax.experimental.pallas import tpu as pltpu


def kernel(
    x,
):
    def body(*refs):
        pass

    out_shape = jax.ShapeDtypeStruct(..., jnp.float32)
    return pl.pallas_call(body, out_shape=out_shape)(...)


# baseline (device time: 367543 ns/iter reference)
import jax
import jax.numpy as jnp
from jax import lax
from jax.experimental import pallas as pl
from jax.experimental.pallas import tpu as pltpu

N_DEV = 8


def kernel(x):
    m_per, n = x.shape

    def body(x_ref, out_ref, send_sems, recv_sems):
        my_pos = lax.axis_index("i")
        left = (my_pos - 1) % N_DEV
        right = (my_pos + 1) % N_DEV

        barrier_sem = pltpu.get_barrier_semaphore()
        for nbr in [left, right]:
            pl.semaphore_signal(
                barrier_sem, inc=1,
                device_id=(nbr,), device_id_type=pl.DeviceIdType.MESH,
            )
        pl.semaphore_wait(barrier_sem, 2)

        out_ref[pl.ds(my_pos * m_per, m_per), :] = x_ref[:, :]

        for h in range(N_DEV - 1):
            send_org = (my_pos - h) % N_DEV
            recv_org = (my_pos - h - 1) % N_DEV
            send = pltpu.make_async_remote_copy(
                src_ref=out_ref.at[pl.ds(send_org * m_per, m_per), :],
                dst_ref=out_ref.at[pl.ds(send_org * m_per, m_per), :],
                send_sem=send_sems.at[h],
                recv_sem=recv_sems.at[h],
                device_id=(right,),
                device_id_type=pl.DeviceIdType.MESH,
            )
            send.start()
            send.wait_send()
            recv = pltpu.make_async_remote_copy(
                src_ref=out_ref.at[pl.ds(recv_org * m_per, m_per), :],
                dst_ref=out_ref.at[pl.ds(recv_org * m_per, m_per), :],
                send_sem=send_sems.at[h],
                recv_sem=recv_sems.at[h],
                device_id=(left,),
                device_id_type=pl.DeviceIdType.MESH,
            )
            recv.wait_recv()

    return pl.pallas_call(
        body,
        out_shape=jax.ShapeDtypeStruct((N_DEV * m_per, n), x.dtype),
        in_specs=[pl.BlockSpec(memory_space=pltpu.VMEM)],
        out_specs=pl.BlockSpec(memory_space=pltpu.VMEM),
        scratch_shapes=[
            pltpu.SemaphoreType.DMA((N_DEV - 1,)),
            pltpu.SemaphoreType.DMA((N_DEV - 1,)),
        ],
        compiler_params=pltpu.CompilerParams(collective_id=0),
    )(x)


# device time: 213040 ns/iter; 1.7252x vs baseline; 1.7252x over previous
import jax
import jax.numpy as jnp
from jax import lax
from jax.experimental import pallas as pl
from jax.experimental.pallas import tpu as pltpu

N_DEV = 8


def kernel(x):
    m_per, n = x.shape
    m_half = m_per // 2

    def body(x_ref, out_ref, send_sems, recv_sems):
        my_pos = lax.axis_index("i")
        left = (my_pos - 1) % N_DEV
        right = (my_pos + 1) % N_DEV

        barrier_sem = pltpu.get_barrier_semaphore()
        for nbr in [left, right]:
            pl.semaphore_signal(
                barrier_sem, inc=1,
                device_id=(nbr,), device_id_type=pl.DeviceIdType.MESH,
            )
        pl.semaphore_wait(barrier_sem, 2)

        out_ref[pl.ds(my_pos * m_per, m_per), :] = x_ref[:, :]

        def half_copy(org, half, sem_idx, dev):
            start = org * m_per + half * m_half
            return pltpu.make_async_remote_copy(
                src_ref=out_ref.at[pl.ds(start, m_half), :],
                dst_ref=out_ref.at[pl.ds(start, m_half), :],
                send_sem=send_sems.at[sem_idx],
                recv_sem=recv_sems.at[sem_idx],
                device_id=(dev,),
                device_id_type=pl.DeviceIdType.MESH,
            )

        for h in range(N_DEV - 1):
            send_r = half_copy((my_pos - h) % N_DEV, 0, 2 * h, right)
            send_l = half_copy((my_pos + h) % N_DEV, 1, 2 * h + 1, left)
            send_r.start()
            send_l.start()
            recv_r = half_copy((my_pos - h - 1) % N_DEV, 0, 2 * h, left)
            recv_l = half_copy((my_pos + h + 1) % N_DEV, 1, 2 * h + 1, right)
            send_r.wait_send()
            send_l.wait_send()
            recv_r.wait_recv()
            recv_l.wait_recv()

    return pl.pallas_call(
        body,
        out_shape=jax.ShapeDtypeStruct((N_DEV * m_per, n), x.dtype),
        in_specs=[pl.BlockSpec(memory_space=pltpu.VMEM)],
        out_specs=pl.BlockSpec(memory_space=pltpu.VMEM),
        scratch_shapes=[
            pltpu.SemaphoreType.DMA((2 * (N_DEV - 1),)),
            pltpu.SemaphoreType.DMA((2 * (N_DEV - 1),)),
        ],
        compiler_params=pltpu.CompilerParams(collective_id=0),
    )(x)


# device time: 201022 ns/iter; 1.8284x vs baseline; 1.0598x over previous
import jax
import jax.numpy as jnp
from jax import lax
from jax.experimental import pallas as pl
from jax.experimental.pallas import tpu as pltpu

N_DEV = 8
SUB = 4


def kernel(x):
    m_per, n = x.shape
    m_half = m_per // 2
    m_sub = m_half // SUB
    n_sems = 2 * (N_DEV - 1) * SUB

    def body(x_ref, out_ref, send_sems, recv_sems):
        my_pos = lax.axis_index("i")
        left = (my_pos - 1) % N_DEV
        right = (my_pos + 1) % N_DEV

        barrier_sem = pltpu.get_barrier_semaphore()
        for nbr in [left, right]:
            pl.semaphore_signal(
                barrier_sem, inc=1,
                device_id=(nbr,), device_id_type=pl.DeviceIdType.MESH,
            )
        pl.semaphore_wait(barrier_sem, 2)

        out_ref[pl.ds(my_pos * m_per, m_per), :] = x_ref[:, :]

        def copy(org, half, h, s, dev):
            row = org * m_per + half * m_half + s * m_sub
            idx = (h * 2 + half) * SUB + s
            return pltpu.make_async_remote_copy(
                src_ref=out_ref.at[pl.ds(row, m_sub), :],
                dst_ref=out_ref.at[pl.ds(row, m_sub), :],
                send_sem=send_sems.at[idx],
                recv_sem=recv_sems.at[idx],
                device_id=(dev,),
                device_id_type=pl.DeviceIdType.MESH,
            )

        def send_r(h, s):
            return copy((my_pos - h) % N_DEV, 0, h, s, right)

        def send_l(h, s):
            return copy((my_pos + h) % N_DEV, 1, h, s, left)

        def recv_r(h, s):
            return copy((my_pos - h - 1) % N_DEV, 0, h, s, left)

        def recv_l(h, s):
            return copy((my_pos + h + 1) % N_DEV, 1, h, s, right)

        for s in range(SUB):
            send_r(0, s).start()
            send_l(0, s).start()
        for h in range(1, N_DEV - 1):
            for s in range(SUB):
                recv_r(h - 1, s).wait_recv()
                send_r(h, s).start()
                recv_l(h - 1, s).wait_recv()
                send_l(h, s).start()
        for s in range(SUB):
            recv_r(N_DEV - 2, s).wait_recv()
            recv_l(N_DEV - 2, s).wait_recv()
        for h in range(N_DEV - 1):
            for s in range(SUB):
                send_r(h, s).wait_send()
                send_l(h, s).wait_send()

    return pl.pallas_call(
        body,
        out_shape=jax.ShapeDtypeStruct((N_DEV * m_per, n), x.dtype),
        in_specs=[pl.BlockSpec(memory_space=pltpu.VMEM)],
        out_specs=pl.BlockSpec(memory_space=pltpu.VMEM),
        scratch_shapes=[
            pltpu.SemaphoreType.DMA((n_sems,)),
            pltpu.SemaphoreType.DMA((n_sems,)),
        ],
        compiler_params=pltpu.CompilerParams(collective_id=0),
    )(x)


# device time: 190044 ns/iter; 1.9340x vs baseline; 1.0578x over previous
import jax
import jax.numpy as jnp
from jax import lax
from jax.experimental import pallas as pl
from jax.experimental.pallas import tpu as pltpu

N_DEV = 8
SUB = 4


def kernel(x):
    m_per, n = x.shape
    m_half = m_per // 2
    m_sub = m_half // SUB
    n_sems = 2 * (N_DEV - 1) * SUB

    def body(x_ref, out_ref, local_sem, send_sems, recv_sems):
        my_pos = lax.axis_index("i")
        left = (my_pos - 1) % N_DEV
        right = (my_pos + 1) % N_DEV

        barrier_sem = pltpu.get_barrier_semaphore()
        for nbr in [left, right]:
            pl.semaphore_signal(
                barrier_sem, inc=1,
                device_id=(nbr,), device_id_type=pl.DeviceIdType.MESH,
            )
        pl.semaphore_wait(barrier_sem, 2)

        local = pltpu.make_async_copy(
            x_ref, out_ref.at[pl.ds(my_pos * m_per, m_per), :], local_sem
        )
        local.start()

        def copy(org, half, h, s, dev, hop0=False):
            loc = half * m_half + s * m_sub
            row = org * m_per + loc
            idx = (h * 2 + half) * SUB + s
            src = x_ref.at[pl.ds(loc, m_sub), :] if hop0 else (
                out_ref.at[pl.ds(row, m_sub), :]
            )
            return pltpu.make_async_remote_copy(
                src_ref=src,
                dst_ref=out_ref.at[pl.ds(row, m_sub), :],
                send_sem=send_sems.at[idx],
                recv_sem=recv_sems.at[idx],
                device_id=(dev,),
                device_id_type=pl.DeviceIdType.MESH,
            )

        def send_r(h, s):
            return copy((my_pos - h) % N_DEV, 0, h, s, right, hop0=(h == 0))

        def send_l(h, s):
            return copy((my_pos + h) % N_DEV, 1, h, s, left, hop0=(h == 0))

        def recv_r(h, s):
            return copy((my_pos - h - 1) % N_DEV, 0, h, s, left)

        def recv_l(h, s):
            return copy((my_pos + h + 1) % N_DEV, 1, h, s, right)

        for s in range(SUB):
            send_r(0, s).start()
            send_l(0, s).start()
        for h in range(1, N_DEV - 1):
            for s in range(SUB):
                recv_r(h - 1, s).wait_recv()
                send_r(h, s).start()
                recv_l(h - 1, s).wait_recv()
                send_l(h, s).start()
        for s in range(SUB):
            recv_r(N_DEV - 2, s).wait_recv()
            recv_l(N_DEV - 2, s).wait_recv()
        local.wait()
        for h in range(N_DEV - 1):
            for s in range(SUB):
                send_r(h, s).wait_send()
                send_l(h, s).wait_send()

    return pl.pallas_call(
        body,
        out_shape=jax.ShapeDtypeStruct((N_DEV * m_per, n), x.dtype),
        in_specs=[pl.BlockSpec(memory_space=pl.ANY)],
        out_specs=pl.BlockSpec(memory_space=pl.ANY),
        scratch_shapes=[
            pltpu.SemaphoreType.DMA,
            pltpu.SemaphoreType.DMA((n_sems,)),
            pltpu.SemaphoreType.DMA((n_sems,)),
        ],
        compiler_params=pltpu.CompilerParams(collective_id=0),
    )(x)


# device time: 138528 ns/iter; 2.6532x vs baseline; 1.3719x over previous
import jax
import jax.numpy as jnp
from jax import lax
from jax.experimental import pallas as pl
from jax.experimental.pallas import tpu as pltpu

N_DEV = 8
PERMS = ((1, 3, 4), (3, 4, 1), (4, 1, 3))
N_PART = 3


def kernel(x):
    m_per, n = x.shape
    base = (m_per // N_PART) // 8 * 8
    part_off = [0, base, 2 * base]
    part_len = [base, base, m_per - 2 * base]

    def body(x_ref, out_ref, local_sem, send_sems, recv_sems):
        my = lax.axis_index("i")

        barrier_sem = pltpu.get_barrier_semaphore()
        for m in (1, 3, 4):
            pl.semaphore_signal(
                barrier_sem, inc=1,
                device_id=(my ^ m,), device_id_type=pl.DeviceIdType.MESH,
            )
        pl.semaphore_wait(barrier_sem, 3)

        local = pltpu.make_async_copy(
            x_ref, out_ref.at[pl.ds(my * m_per, m_per), :], local_sem
        )
        local.start()

        def rc(p, idx, origin, partner, from_x):
            off, ln = part_off[p], part_len[p]
            dst = out_ref.at[pl.ds(origin * m_per + off, ln), :]
            src = x_ref.at[pl.ds(off, ln), :] if from_x else dst
            return pltpu.make_async_remote_copy(
                src_ref=src,
                dst_ref=dst,
                send_sem=send_sems.at[p * 7 + idx],
                recv_sem=recv_sems.at[p * 7 + idx],
                device_id=(partner,),
                device_id_type=pl.DeviceIdType.MESH,
            )


        for p, (m1, _, _) in enumerate(PERMS):
            rc(p, 0, my, my ^ m1, True).start()
        for p, (m1, m2, _) in enumerate(PERMS):
            rc(p, 0, my ^ m1, my ^ m1, False).wait_recv()
            rc(p, 1, my, my ^ m2, True).start()
            rc(p, 2, my ^ m1, my ^ m2, False).start()
        for p, (m1, _, m3) in enumerate(PERMS):
            rc(p, 3, my, my ^ m3, True).start()
            rc(p, 4, my ^ m1, my ^ m3, False).start()
        for p, (m1, m2, m3) in enumerate(PERMS):
            rc(p, 1, my ^ m2, my ^ m2, False).wait_recv()
            rc(p, 2, my ^ m1 ^ m2, my ^ m2, False).wait_recv()
            rc(p, 5, my ^ m2, my ^ m3, False).start()
            rc(p, 6, my ^ m1 ^ m2, my ^ m3, False).start()
        for p, (m1, m2, m3) in enumerate(PERMS):
            rc(p, 3, my ^ m3, my ^ m3, False).wait_recv()
            rc(p, 4, my ^ m1 ^ m3, my ^ m3, False).wait_recv()
            rc(p, 5, my ^ m2 ^ m3, my ^ m3, False).wait_recv()
            rc(p, 6, my ^ m1 ^ m2 ^ m3, my ^ m3, False).wait_recv()
        local.wait()
        for p in range(N_PART):
            for idx in range(7):
                rc(p, idx, my, my, idx in (0, 1, 3)).wait_send()

    return pl.pallas_call(
        body,
        out_shape=jax.ShapeDtypeStruct((N_DEV * m_per, n), x.dtype),
        in_specs=[pl.BlockSpec(memory_space=pl.ANY)],
        out_specs=pl.BlockSpec(memory_space=pl.ANY),
        scratch_shapes=[
            pltpu.SemaphoreType.DMA,
            pltpu.SemaphoreType.DMA((N_PART * 7,)),
            pltpu.SemaphoreType.DMA((N_PART * 7,)),
        ],
        compiler_params=pltpu.CompilerParams(collective_id=0),
    )(x)


# device time: 136461 ns/iter; 2.6934x vs baseline; 1.0151x over previous
import jax
import jax.numpy as jnp
from jax import lax
from jax.experimental import pallas as pl
from jax.experimental.pallas import tpu as pltpu

N_DEV = 8
PERMS = ((1, 3, 4), (3, 4, 1), (4, 1, 3))
N_PART = 3


def kernel(x):
    m_per, n = x.shape
    base = (m_per // N_PART) // 8 * 8
    part_off = [0, base, 2 * base]
    part_len = [base, base, m_per - 2 * base]

    def body(x_ref, out_ref, local_sem, send_sems, recv_sems):
        my = lax.axis_index("i")

        barrier_sem = pltpu.get_barrier_semaphore()
        for m in (1, 3, 4):
            pl.semaphore_signal(
                barrier_sem, inc=1,
                device_id=(my ^ m,), device_id_type=pl.DeviceIdType.MESH,
            )
        pl.semaphore_wait(barrier_sem, 3)

        local = pltpu.make_async_copy(
            x_ref, out_ref.at[pl.ds(my * m_per, m_per), :], local_sem
        )
        local.start()

        def rc(p, idx, origin, partner, from_x):
            off, ln = part_off[p], part_len[p]
            dst = out_ref.at[pl.ds(origin * m_per + off, ln), :]
            src = x_ref.at[pl.ds(off, ln), :] if from_x else dst
            return pltpu.make_async_remote_copy(
                src_ref=src,
                dst_ref=dst,
                send_sem=send_sems.at[p * 7 + idx],
                recv_sem=recv_sems.at[p * 7 + idx],
                device_id=(partner,),
                device_id_type=pl.DeviceIdType.MESH,
            )


        for p, (m1, _, _) in enumerate(PERMS):
            rc(p, 0, my, my ^ m1, True).start()
        for p, (_, m2, _) in enumerate(PERMS):
            rc(p, 1, my, my ^ m2, True).start()
        for p, (_, _, m3) in enumerate(PERMS):
            rc(p, 3, my, my ^ m3, True).start()
        for p, (m1, m2, _) in enumerate(PERMS):
            rc(p, 0, my ^ m1, my ^ m1, False).wait_recv()
            rc(p, 2, my ^ m1, my ^ m2, False).start()
        for p, (m1, _, m3) in enumerate(PERMS):
            rc(p, 4, my ^ m1, my ^ m3, False).start()
        for p, (m1, m2, m3) in enumerate(PERMS):
            rc(p, 1, my ^ m2, my ^ m2, False).wait_recv()
            rc(p, 2, my ^ m1 ^ m2, my ^ m2, False).wait_recv()
            rc(p, 5, my ^ m2, my ^ m3, False).start()
            rc(p, 6, my ^ m1 ^ m2, my ^ m3, False).start()
        for p, (m1, m2, m3) in enumerate(PERMS):
            rc(p, 3, my ^ m3, my ^ m3, False).wait_recv()
            rc(p, 4, my ^ m1 ^ m3, my ^ m3, False).wait_recv()
            rc(p, 5, my ^ m2 ^ m3, my ^ m3, False).wait_recv()
            rc(p, 6, my ^ m1 ^ m2 ^ m3, my ^ m3, False).wait_recv()
        local.wait()
        for p in range(N_PART):
            for idx in range(7):
                rc(p, idx, my, my, idx in (0, 1, 3)).wait_send()

    return pl.pallas_call(
        body,
        out_shape=jax.ShapeDtypeStruct((N_DEV * m_per, n), x.dtype),
        in_specs=[pl.BlockSpec(memory_space=pl.ANY)],
        out_specs=pl.BlockSpec(memory_space=pl.ANY),
        scratch_shapes=[
            pltpu.SemaphoreType.DMA,
            pltpu.SemaphoreType.DMA((N_PART * 7,)),
            pltpu.SemaphoreType.DMA((N_PART * 7,)),
        ],
        compiler_params=pltpu.CompilerParams(collective_id=0),
    )(x)
